# baseline (speedup 1.0000x reference)
import jax
import jax.numpy as jnp
from jax.experimental import pallas as pl
from jax.experimental.pallas import tpu as pltpu

_BM = 256
_NBUF = 3


def _dot_t(w, x):
    return jax.lax.dot_general(
        w, x,
        dimension_numbers=(((1,), (1,)), ((), ())),
        preferred_element_type=jnp.float32,
    )


def _pathcon_body(x_any, w_any, b_ref, scores_any, sig_any,
                  wbuf, x0, x1, x2, s0, s1, s2, g0, g1, g2,
                  w_sem, in_sem, out_sem):
    xb = [x0, x1, x2]
    sb = [s0, s1, s2]
    gb = [g0, g1, g2]
    batch = x_any.shape[0]
    nblk = batch // _BM

    def in_copy(i, j):
        return pltpu.make_async_copy(
            x_any.at[pl.ds(i * _BM, _BM), :], xb[j], in_sem.at[j])

    def s_copy(i, j):
        return pltpu.make_async_copy(
            sb[j], scores_any.at[:, pl.ds(i * _BM, _BM)], out_sem.at[j, 0])

    def g_copy(i, j):
        return pltpu.make_async_copy(
            gb[j], sig_any.at[:, pl.ds(i * _BM, _BM)], out_sem.at[j, 1])

    w_copy = pltpu.make_async_copy(w_any, wbuf, w_sem)
    w_copy.start()
    for j in range(_NBUF):
        in_copy(j, j).start()
    w_copy.wait()

    def step(i, j, do_start, do_wait_out):
        in_copy(i, j).wait()
        acc = _dot_t(wbuf[...], xb[j][...])
        if do_start:
            in_copy(i + _NBUF, j).start()
        if do_wait_out:
            s_copy(i - _NBUF, j).wait()
            g_copy(i - _NBUF, j).wait()
        scores = acc + b_ref[...]
        sb[j][...] = scores
        gb[j][...] = jax.nn.sigmoid(scores)
        s_copy(i, j).start()
        g_copy(i, j).start()

    for j in range(_NBUF):
        step(j, j, do_start=True, do_wait_out=False)

    def outer(g, carry):
        for j in range(_NBUF):
            step(g * _NBUF + j, j, do_start=True, do_wait_out=True)
        return carry

    jax.lax.fori_loop(1, nblk // _NBUF - 1, outer, 0)

    rem_start = (nblk // _NBUF - 1) * _NBUF
    for i in range(rem_start, nblk):
        j = i % _NBUF
        step(i, j, do_start=(i + _NBUF < nblk), do_wait_out=True)

    for i in range(nblk - _NBUF, nblk):
        j = i % _NBUF
        s_copy(i, j).wait()
        g_copy(i, j).wait()


def kernel(path_features, labels, W, b):
    del labels
    batch, n_paths = path_features.shape
    n_rel = W.shape[0]
    b2 = b.reshape(n_rel, 1)

    out_shape = [
        jax.ShapeDtypeStruct((n_rel, batch), jnp.float32),
        jax.ShapeDtypeStruct((n_rel, batch), jnp.float32),
    ]
    scores_t, sig_t = pl.pallas_call(
        _pathcon_body,
        in_specs=[
            pl.BlockSpec(memory_space=pl.ANY),
            pl.BlockSpec(memory_space=pl.ANY),
            pl.BlockSpec((n_rel, 1), lambda: (0, 0)),
        ],
        out_specs=[
            pl.BlockSpec(memory_space=pl.ANY),
            pl.BlockSpec(memory_space=pl.ANY),
        ],
        out_shape=out_shape,
        compiler_params=pltpu.CompilerParams(
            vmem_limit_bytes=100 * 1024 * 1024,
        ),
        scratch_shapes=(
            [pltpu.VMEM((n_rel, n_paths), jnp.float32)]
            + [pltpu.VMEM((_BM, n_paths), jnp.float32) for _ in range(_NBUF)]
            + [pltpu.VMEM((n_rel, _BM), jnp.float32) for _ in range(2 * _NBUF)]
            + [
                pltpu.SemaphoreType.DMA,
                pltpu.SemaphoreType.DMA((_NBUF,)),
                pltpu.SemaphoreType.DMA((_NBUF, 2)),
            ]
        ),
    )(path_features, W, b2)
    return (jnp.swapaxes(scores_t, 0, 1), jnp.swapaxes(sig_t, 0, 1))


# speedup vs baseline: 1.2290x; 1.2290x over previous
import jax
import jax.numpy as jnp
from jax.experimental import pallas as pl
from jax.experimental.pallas import tpu as pltpu

_BM = 256


def _pathcon_body(x_ref, w_ref, b_ref, scores_ref, sig_ref, acc_ref):
    i = pl.program_id(0)
    n = pl.num_programs(0)

    @pl.when(i > 0)
    def _epilogue():
        scores = acc_ref[(i - 1) % 2] + b_ref[...]
        scores_ref[...] = scores
        sig_ref[...] = jax.nn.sigmoid(scores)

    @pl.when(i < n - 1)
    def _matmul():
        acc_ref[i % 2] = jax.lax.dot_general(
            w_ref[...], x_ref[...],
            dimension_numbers=(((1,), (1,)), ((), ())),
            preferred_element_type=jnp.float32,
        )


def kernel(path_features, labels, W, b):
    del labels
    batch, n_paths = path_features.shape
    n_rel = W.shape[0]
    b2 = b.reshape(n_rel, 1)

    nblk = batch // _BM
    grid = (nblk + 1,)
    out_shape = [
        jax.ShapeDtypeStruct((n_rel, batch), jnp.float32),
        jax.ShapeDtypeStruct((n_rel, batch), jnp.float32),
    ]
    scores_t, sig_t = pl.pallas_call(
        _pathcon_body,
        grid=grid,
        in_specs=[
            pl.BlockSpec((_BM, n_paths), lambda i: (jnp.minimum(i, nblk - 1), 0)),
            pl.BlockSpec((n_rel, n_paths), lambda i: (0, 0)),
            pl.BlockSpec((n_rel, 1), lambda i: (0, 0)),
        ],
        out_specs=[
            pl.BlockSpec((n_rel, _BM), lambda i: (0, jnp.maximum(i - 1, 0))),
            pl.BlockSpec((n_rel, _BM), lambda i: (0, jnp.maximum(i - 1, 0))),
        ],
        out_shape=out_shape,
        scratch_shapes=[pltpu.VMEM((2, n_rel, _BM), jnp.float32)],
        compiler_params=pltpu.CompilerParams(
            dimension_semantics=("arbitrary",),
        ),
    )(path_features, W, b2)
    return (jnp.swapaxes(scores_t, 0, 1), jnp.swapaxes(sig_t, 0, 1))


# speedup vs baseline: 1.2471x; 1.0147x over previous
import jax
import jax.numpy as jnp
from jax.experimental import pallas as pl
from jax.experimental.pallas import tpu as pltpu

_BM = 256


def _pathcon_body(x_ref, w_ref, b_ref, scores_ref, sig_ref):
    acc = jax.lax.dot_general(
        w_ref[...], x_ref[...],
        dimension_numbers=(((1,), (1,)), ((), ())),
        preferred_element_type=jnp.float32,
    )
    scores = acc + b_ref[...]
    scores_ref[...] = scores
    sig_ref[...] = jax.nn.sigmoid(scores)


def kernel(path_features, labels, W, b):
    del labels
    batch, n_paths = path_features.shape
    n_rel = W.shape[0]
    b2 = b.reshape(n_rel, 1)

    grid = (batch // _BM,)
    out_shape = [
        jax.ShapeDtypeStruct((n_rel, batch), jnp.float32),
        jax.ShapeDtypeStruct((n_rel, batch), jnp.float32),
    ]
    scores_t, sig_t = pl.pallas_call(
        _pathcon_body,
        grid=grid,
        in_specs=[
            pl.BlockSpec((_BM, n_paths), lambda i: (i, 0)),
            pl.BlockSpec((n_rel, n_paths), lambda i: (0, 0)),
            pl.BlockSpec((n_rel, 1), lambda i: (0, 0)),
        ],
        out_specs=[
            pl.BlockSpec((n_rel, _BM), lambda i: (0, i)),
            pl.BlockSpec((n_rel, _BM), lambda i: (0, i)),
        ],
        out_shape=out_shape,
        compiler_params=pltpu.CompilerParams(
            dimension_semantics=("arbitrary",),
        ),
    )(path_features, W, b2)
    return (jnp.swapaxes(scores_t, 0, 1), jnp.swapaxes(sig_t, 0, 1))
